# baseline (speedup 1.0000x reference)
import functools

import jax
import jax.numpy as jnp
from jax import lax
from jax.experimental import pallas as pl
from jax.experimental.pallas import tpu as pltpu
from jax.experimental.pallas import tpu_sc as plsc

B, T = 8, 2048
D_MODEL = 1024
N_L1 = 8
N_L2 = 1024
BD = 256
TB = 512
NT = T // TB

NW = 32
RPW = (B * T) // NW
CHUNK = 128
NCHUNK = RPW // CHUNK
WPB = T // RPW

TB3 = 1024


def _ln(x, g, b, eps=1e-5):
    m = jnp.mean(x, axis=-1, keepdims=True)
    v = jnp.mean((x - m) ** 2, axis=-1, keepdims=True)
    return (x - m) / jnp.sqrt(v + eps) * g + b


def _stage1_body(idx_ref, temp_ref, x_ref, cb_ref, W1_ref, b1_ref, g1_ref,
                 bt1_ref, hard_ref, soft_ref, gidx_ref):
    x = x_ref[0]
    cb = cb_ref[0]
    temp = temp_ref[0]

    h0 = jnp.dot(x, W1_ref[...], preferred_element_type=jnp.float32) + b1_ref[...]
    h = _ln(h0, g1_ref[...], bt1_ref[...])
    hn = h / jnp.maximum(jnp.sqrt(jnp.sum(h * h, axis=-1, keepdims=True)), 1e-12)

    cb_inv = 1.0 / jnp.maximum(
        jnp.sqrt(jnp.sum(cb * cb, axis=-1, keepdims=True)), 1e-12)
    cbn = cb * cb_inv

    logits = jnp.dot(hn, cbn.T, preferred_element_type=jnp.float32) / temp

    rowmax = jnp.max(logits, axis=-1, keepdims=True)
    e = jnp.exp(logits - rowmax)
    soft_ref[0] = e / jnp.sum(e, axis=-1, keepdims=True)

    kiota = jax.lax.broadcasted_iota(jnp.int32, logits.shape, 1)
    idx = jnp.min(jnp.where(logits == rowmax, kiota, N_L2), axis=-1,
                  keepdims=True)
    hard_ref[0, 0] = idx.T.astype(jnp.int32)
    b = pl.program_id(0)
    gidx_ref[0, 0] = (idx.T + idx_ref[b] * N_L2).astype(jnp.int32)


def _stage1(local_prosody, codebooks, W1, b1, g1, bt1, l1_indices, temperature):
    grid_spec = pltpu.PrefetchScalarGridSpec(
        num_scalar_prefetch=1,
        grid=(B, NT),
        in_specs=[
            pl.BlockSpec(memory_space=pltpu.SMEM),
            pl.BlockSpec((1, TB, D_MODEL), lambda b, t, i: (b, t, 0)),
            pl.BlockSpec((1, N_L2, BD), lambda b, t, i: (i[b], 0, 0)),
            pl.BlockSpec((D_MODEL, BD), lambda b, t, i: (0, 0)),
            pl.BlockSpec((BD,), lambda b, t, i: (0,)),
            pl.BlockSpec((BD,), lambda b, t, i: (0,)),
            pl.BlockSpec((BD,), lambda b, t, i: (0,)),
        ],
        out_specs=[
            pl.BlockSpec((1, 1, 1, TB), lambda b, t, i: (b, t, 0, 0)),
            pl.BlockSpec((1, TB, N_L2), lambda b, t, i: (b, t, 0)),
            pl.BlockSpec((1, 1, 1, TB), lambda b, t, i: (b, t, 0, 0)),
        ],
    )
    hard4, soft, gidx4 = pl.pallas_call(
        _stage1_body,
        grid_spec=grid_spec,
        out_shape=[
            jax.ShapeDtypeStruct((B, NT, 1, TB), jnp.int32),
            jax.ShapeDtypeStruct((B, T, N_L2), jnp.float32),
            jax.ShapeDtypeStruct((B, NT, 1, TB), jnp.int32),
        ],
    )(l1_indices.astype(jnp.int32),
      jnp.reshape(jnp.asarray(temperature, jnp.float32), (1,)),
      local_prosody, codebooks, W1, b1, g1, bt1)
    return hard4.reshape(B, T), soft, gidx4.reshape(NW, NCHUNK, CHUNK)


@functools.partial(
    pl.kernel,
    out_type=jax.ShapeDtypeStruct((B * T, BD), jnp.float32),
    mesh=plsc.VectorSubcoreMesh(core_axis_name="c", subcore_axis_name="s"),
    scratch_types=[
        pltpu.VMEM((NCHUNK, CHUNK), jnp.int32),
        pltpu.VMEM((2, CHUNK, BD), jnp.float32),
        pltpu.SemaphoreType.DMA,
        pltpu.SemaphoreType.DMA,
    ],
)
def _sc_gather(table_hbm, gidx_hbm, out_hbm, idx_v, buf_v, sem0, sem1):
    wid = lax.axis_index("c") * 16 + lax.axis_index("s")
    base = wid * RPW

    pltpu.sync_copy(gidx_hbm.at[wid], idx_v)

    sems = (sem0, sem1)
    copies = [None, None]
    for c in range(NCHUNK):
        ph = c % 2
        if copies[ph] is not None:
            copies[ph].wait()
            pltpu.sync_copy(buf_v.at[ph],
                            out_hbm.at[pl.ds(base + (c - 2) * CHUNK, CHUNK)])
        copies[ph] = pltpu.async_copy(table_hbm.at[idx_v.at[c]],
                                      buf_v.at[ph], sems[ph])
    for c in range(NCHUNK - 2, NCHUNK):
        ph = c % 2
        copies[ph].wait()
        pltpu.sync_copy(buf_v.at[ph],
                        out_hbm.at[pl.ds(base + c * CHUNK, CHUNK)])


def _stage3_body(e_ref, W2_ref, b2_ref, g2_ref, bt2_ref, out_ref):
    e0 = jnp.dot(e_ref[...], W2_ref[...],
                 preferred_element_type=jnp.float32) + b2_ref[...]
    out_ref[...] = _ln(e0, g2_ref[...], bt2_ref[...])


def _stage3(emb_low, W2, b2, g2, bt2):
    return pl.pallas_call(
        _stage3_body,
        grid=(B * T // TB3,),
        in_specs=[
            pl.BlockSpec((TB3, BD), lambda t: (t, 0)),
            pl.BlockSpec((BD, D_MODEL), lambda t: (0, 0)),
            pl.BlockSpec((D_MODEL,), lambda t: (0,)),
            pl.BlockSpec((D_MODEL,), lambda t: (0,)),
            pl.BlockSpec((D_MODEL,), lambda t: (0,)),
        ],
        out_specs=pl.BlockSpec((TB3, D_MODEL), lambda t: (t, 0)),
        out_shape=jax.ShapeDtypeStruct((B * T, D_MODEL), jnp.float32),
    )(emb_low, W2, b2, g2, bt2)


@jax.jit
def _run(local_prosody, codebooks, W1, b1, g1, bt1, W2, b2, g2, bt2,
         l1_indices, temperature):
    hard, soft, gidx = _stage1(local_prosody, codebooks, W1, b1, g1, bt1,
                               l1_indices, temperature)
    table = codebooks.reshape(N_L1 * N_L2, BD)
    emb_low = _sc_gather(table, gidx)
    emb = _stage3(emb_low, W2, b2, g2, bt2)
    return (hard, soft, emb.reshape(B, T, D_MODEL),
            emb_low.reshape(B, T, BD))


def kernel(local_prosody, codebooks, W1, b1, g1, bt1, W2, b2, g2, bt2,
           l1_indices, temperature):
    return _run(local_prosody, codebooks, W1, b1, g1, bt1, W2, b2, g2, bt2,
                l1_indices, temperature)
